# baseline (speedup 1.0000x reference)
import functools

import jax
import jax.numpy as jnp
from jax import lax
from jax.experimental import pallas as pl
from jax.experimental.pallas import tpu as pltpu
from jax.experimental.pallas import tpu_sc as plsc

BS = 32
NCROPS = 10
T = 32
FVIS = 1024
FTXT = 512
FFUSE = FVIS + FTXT
K = T // 10


VPB = 2


def _mlp_mag_kernel(x_ref, t_ref, w1_ref, b1_ref, w2_ref, b2_ref, w3_ref,
                    b3_ref, scores_ref, mags_ref, idx_ref, sel_ref):
    rows = VPB * NCROPS * T
    xv = x_ref[...].reshape(rows, FVIS)
    xt = t_ref[...].reshape(rows, FTXT)
    h = jnp.dot(xv.astype(jnp.bfloat16), w1_ref[:FVIS, :].astype(jnp.bfloat16),
                preferred_element_type=jnp.float32)
    h += jnp.dot(xt.astype(jnp.bfloat16), w1_ref[FVIS:, :].astype(jnp.bfloat16),
                 preferred_element_type=jnp.float32)
    h = jax.nn.relu(h + b1_ref[0])
    h2 = jax.nn.relu(
        jnp.dot(h, w2_ref[...], preferred_element_type=jnp.float32) + b2_ref[0])
    logit = jnp.dot(h2, w3_ref[...], preferred_element_type=jnp.float32)
    s = jax.nn.sigmoid(logit + b3_ref[0])
    sblk = s.reshape(VPB, NCROPS, T, 1).mean(axis=1)
    scores_ref[...] = sblk

    sq = (xv * xv).sum(axis=1, keepdims=True) + (xt * xt).sum(
        axis=1, keepdims=True)
    mblk = jnp.sqrt(sq).reshape(VPB, NCROPS, T, 1).mean(axis=1)
    mags_ref[...] = mblk

    tio = jax.lax.broadcasted_iota(jnp.int32, (T, 1), 0)
    for g in range(VPB):
        m = mblk[g]
        sg = sblk[g]
        ssum = jnp.float32(0.0)
        for kk in range(K):
            val = jnp.max(m)
            pos = jnp.min(jnp.where(m == val, tio, T))
            idx_ref[g, 0, kk] = pos
            hit = tio == pos
            ssum += jnp.sum(jnp.where(hit, sg, 0.0))
            m = jnp.where(hit, -jnp.inf, m)
        sel_ref[g, 0, 0] = ssum * (1.0 / K)


def _scores_mags(inputs, text, W1, b1, W2, b2, W3, b3):
    return pl.pallas_call(
        _mlp_mag_kernel,
        grid=(BS // VPB,),
        in_specs=[
            pl.BlockSpec((VPB, NCROPS, T, FVIS), lambda b: (b, 0, 0, 0)),
            pl.BlockSpec((VPB, NCROPS, T, FTXT), lambda b: (b, 0, 0, 0)),
            pl.BlockSpec((FFUSE, 512), lambda b: (0, 0)),
            pl.BlockSpec((1, 512), lambda b: (0, 0)),
            pl.BlockSpec((512, 128), lambda b: (0, 0)),
            pl.BlockSpec((1, 128), lambda b: (0, 0)),
            pl.BlockSpec((128, 1), lambda b: (0, 0)),
            pl.BlockSpec((1, 1), lambda b: (0, 0)),
        ],
        out_specs=[
            pl.BlockSpec((VPB, T, 1), lambda b: (b, 0, 0)),
            pl.BlockSpec((VPB, T, 1), lambda b: (b, 0, 0)),
            pl.BlockSpec((VPB, 1, K), lambda b: (b, 0, 0),
                         memory_space=pltpu.SMEM),
            pl.BlockSpec((VPB, 1, 1), lambda b: (b, 0, 0),
                         memory_space=pltpu.SMEM),
        ],
        out_shape=[
            jax.ShapeDtypeStruct((BS, T, 1), jnp.float32),
            jax.ShapeDtypeStruct((BS, T, 1), jnp.float32),
            jax.ShapeDtypeStruct((BS, 1, K), jnp.int32),
            jax.ShapeDtypeStruct((BS, 1, 1), jnp.float32),
        ],
    )(inputs, text, W1, b1.reshape(1, 512), W2, b2.reshape(1, 128), W3,
      b3.reshape(1, 1))


HALF = BS // 2
NROWS = NCROPS * HALF * K
LANES = 16


def _sc_gather_body(vis_hbm, txt_hbm, idx_hbm, out_n, out_a0, out_a1, out_a2,
                    out_a3, out_a4, idx_v, vbuf_n, tbuf_n, vbuf_a, tbuf_a,
                    sem, wsem):
    cid = lax.axis_index("c")
    sid = lax.axis_index("s")
    wid = sid * 2 + cid

    @pl.when(wid < K * NCROPS)
    def _():
        pltpu.sync_copy(idx_hbm, idx_v)

        lane = lax.iota(jnp.int32, LANES)
        j = wid // NCROPS
        c = lax.rem(wid, NCROPS)
        base = wid * LANES
        zero = jnp.zeros((LANES,), jnp.int32)

        copies = []
        for off, vbuf, tbuf in ((0, vbuf_n, tbuf_n), (HALF, vbuf_a, tbuf_a)):
            video = off + lane
            t = plsc.load_gather(idx_v, [video, zero, zero + j])
            src = video * (NCROPS * T) + c * T + t
            copies.append(pltpu.async_copy(vis_hbm.at[src], vbuf, sem))
            copies.append(pltpu.async_copy(txt_hbm.at[src], tbuf, sem))
        for cp in copies:
            cp.wait()

        writes = []
        for out, vbuf, tbuf in ((out_n, vbuf_n, tbuf_n),
                                (out_a0, vbuf_a, tbuf_a),
                                (out_a1, vbuf_a, tbuf_a),
                                (out_a2, vbuf_a, tbuf_a),
                                (out_a3, vbuf_a, tbuf_a),
                                (out_a4, vbuf_a, tbuf_a)):
            writes.append(
                pltpu.async_copy(vbuf,
                                 out.at[pl.ds(base, LANES), pl.ds(0, FVIS)],
                                 wsem))
            writes.append(
                pltpu.async_copy(tbuf,
                                 out.at[pl.ds(base, LANES),
                                        pl.ds(FVIS, FTXT)], wsem))
        for w in writes:
            w.wait()


_sc_gather = functools.partial(
    pl.kernel,
    mesh=plsc.VectorSubcoreMesh(core_axis_name="c", subcore_axis_name="s"),
    compiler_params=pltpu.CompilerParams(needs_layout_passes=False),
    out_type=[jax.ShapeDtypeStruct((NROWS, FFUSE), jnp.float32)] * 6,
    scratch_types=[
        pltpu.VMEM((BS, 1, K), jnp.int32),
        pltpu.VMEM((LANES, FVIS), jnp.float32),
        pltpu.VMEM((LANES, FTXT), jnp.float32),
        pltpu.VMEM((LANES, FVIS), jnp.float32),
        pltpu.VMEM((LANES, FTXT), jnp.float32),
        pltpu.SemaphoreType.DMA,
        pltpu.SemaphoreType.DMA,
    ],
)(_sc_gather_body)


def kernel(inputs, text, W1, b1, W2, b2, W3, b3):
    scores, mags3, idx, sel = _scores_mags(inputs, text, W1, b1, W2, b2, W3,
                                           b3)
    mags = mags3[:, :, 0]
    score_normal = sel[:HALF, 0]
    score_abnormal = sel[HALF:, 0]

    vis2 = inputs.reshape(BS * NCROPS * T, FVIS)
    txt2 = text.reshape(BS * NCROPS * T, FTXT)
    feat_n, feat_a0, feat_a1, feat_a2, feat_a3, feat_a4 = _sc_gather(
        vis2, txt2, idx)

    def shape(f):
        return f.reshape(K, NCROPS * HALF, FFUSE).transpose(1, 0, 2)

    return (score_abnormal, score_normal, shape(feat_a0), shape(feat_n),
            shape(feat_a1), shape(feat_a2), scores, shape(feat_a3),
            shape(feat_a4), mags)


# speedup vs baseline: 1.2176x; 1.2176x over previous
import functools

import jax
import jax.numpy as jnp
from jax import lax
from jax.experimental import pallas as pl
from jax.experimental.pallas import tpu as pltpu
from jax.experimental.pallas import tpu_sc as plsc

BS = 32
NCROPS = 10
T = 32
FVIS = 1024
FTXT = 512
FFUSE = FVIS + FTXT
K = T // 10


VPB = 4


def _mlp_mag_kernel(x_ref, t_ref, w1_ref, b1_ref, w2_ref, b2_ref, w3_ref,
                    b3_ref, scores_ref, mags_ref, idx_ref, sel_ref):
    rows = VPB * NCROPS * T
    xv = x_ref[...].reshape(rows, FVIS)
    xt = t_ref[...].reshape(rows, FTXT)
    h = jnp.dot(xv.astype(jnp.bfloat16), w1_ref[:FVIS, :].astype(jnp.bfloat16),
                preferred_element_type=jnp.float32)
    h += jnp.dot(xt.astype(jnp.bfloat16), w1_ref[FVIS:, :].astype(jnp.bfloat16),
                 preferred_element_type=jnp.float32)
    h = jax.nn.relu(h + b1_ref[0])
    h2 = jax.nn.relu(
        jnp.dot(h, w2_ref[...], preferred_element_type=jnp.float32) + b2_ref[0])
    logit = jnp.dot(h2, w3_ref[...], preferred_element_type=jnp.float32)
    s = jax.nn.sigmoid(logit + b3_ref[0])
    sblk = s.reshape(VPB, NCROPS, T, 1).mean(axis=1)
    scores_ref[...] = sblk

    sq = (xv * xv).sum(axis=1, keepdims=True) + (xt * xt).sum(
        axis=1, keepdims=True)
    mblk = jnp.sqrt(sq).reshape(VPB, NCROPS, T, 1).mean(axis=1)
    mags_ref[...] = mblk

    tio = jax.lax.broadcasted_iota(jnp.int32, (VPB, T, 1), 1)
    m = mblk
    ssum = jnp.zeros((VPB, 1, 1), jnp.float32)
    poss = []
    for kk in range(K):
        val = jnp.max(m, axis=1, keepdims=True)
        pos = jnp.min(jnp.where(m == val, tio, T), axis=1, keepdims=True)
        poss.append(pos)
        hit = tio == pos
        ssum += jnp.sum(jnp.where(hit, sblk, 0.0), axis=1, keepdims=True)
        m = jnp.where(hit, -jnp.inf, m)
    idx_ref[...] = jnp.concatenate(poss, axis=2)
    sel_ref[...] = ssum * (1.0 / K)


def _scores_mags(inputs, text, W1, b1, W2, b2, W3, b3):
    return pl.pallas_call(
        _mlp_mag_kernel,
        grid=(BS // VPB,),
        in_specs=[
            pl.BlockSpec((VPB, NCROPS, T, FVIS), lambda b: (b, 0, 0, 0)),
            pl.BlockSpec((VPB, NCROPS, T, FTXT), lambda b: (b, 0, 0, 0)),
            pl.BlockSpec((FFUSE, 512), lambda b: (0, 0)),
            pl.BlockSpec((1, 512), lambda b: (0, 0)),
            pl.BlockSpec((512, 128), lambda b: (0, 0)),
            pl.BlockSpec((1, 128), lambda b: (0, 0)),
            pl.BlockSpec((128, 1), lambda b: (0, 0)),
            pl.BlockSpec((1, 1), lambda b: (0, 0)),
        ],
        out_specs=[
            pl.BlockSpec((VPB, T, 1), lambda b: (b, 0, 0)),
            pl.BlockSpec((VPB, T, 1), lambda b: (b, 0, 0)),
            pl.BlockSpec((VPB, 1, K), lambda b: (b, 0, 0)),
            pl.BlockSpec((VPB, 1, 1), lambda b: (b, 0, 0)),
        ],
        out_shape=[
            jax.ShapeDtypeStruct((BS, T, 1), jnp.float32),
            jax.ShapeDtypeStruct((BS, T, 1), jnp.float32),
            jax.ShapeDtypeStruct((BS, 1, K), jnp.int32),
            jax.ShapeDtypeStruct((BS, 1, 1), jnp.float32),
        ],
    )(inputs, text, W1, b1.reshape(1, 512), W2, b2.reshape(1, 128), W3,
      b3.reshape(1, 1))


HALF = BS // 2
NROWS = NCROPS * HALF * K
LANES = 16


def _sc_gather_body(vis_hbm, txt_hbm, idx_hbm, out_n, out_a0, out_a1, out_a2,
                    out_a3, out_a4, idx_v, vbuf_n, tbuf_n, vbuf_a, tbuf_a,
                    sem, wsem):
    cid = lax.axis_index("c")
    sid = lax.axis_index("s")
    wid = sid * 2 + cid

    @pl.when(wid < K * NCROPS)
    def _():
        pltpu.sync_copy(idx_hbm, idx_v)

        lane = lax.iota(jnp.int32, LANES)
        j = wid // NCROPS
        c = lax.rem(wid, NCROPS)
        base = wid * LANES
        zero = jnp.zeros((LANES,), jnp.int32)

        copies = []
        for off, vbuf, tbuf in ((0, vbuf_n, tbuf_n), (HALF, vbuf_a, tbuf_a)):
            video = off + lane
            t = plsc.load_gather(idx_v, [video, zero, zero + j])
            src = video * (NCROPS * T) + c * T + t
            copies.append(pltpu.async_copy(vis_hbm.at[src], vbuf, sem))
            copies.append(pltpu.async_copy(txt_hbm.at[src], tbuf, sem))
        for cp in copies:
            cp.wait()

        writes = []
        for out, vbuf, tbuf in ((out_n, vbuf_n, tbuf_n),
                                (out_a0, vbuf_a, tbuf_a),
                                (out_a1, vbuf_a, tbuf_a),
                                (out_a2, vbuf_a, tbuf_a),
                                (out_a3, vbuf_a, tbuf_a),
                                (out_a4, vbuf_a, tbuf_a)):
            writes.append(
                pltpu.async_copy(vbuf,
                                 out.at[pl.ds(base, LANES), pl.ds(0, FVIS)],
                                 wsem))
            writes.append(
                pltpu.async_copy(tbuf,
                                 out.at[pl.ds(base, LANES),
                                        pl.ds(FVIS, FTXT)], wsem))
        for w in writes:
            w.wait()


_sc_gather = functools.partial(
    pl.kernel,
    mesh=plsc.VectorSubcoreMesh(core_axis_name="c", subcore_axis_name="s"),
    compiler_params=pltpu.CompilerParams(needs_layout_passes=False),
    out_type=[jax.ShapeDtypeStruct((NROWS, FFUSE), jnp.float32)] * 6,
    scratch_types=[
        pltpu.VMEM((BS, 1, K), jnp.int32),
        pltpu.VMEM((LANES, FVIS), jnp.float32),
        pltpu.VMEM((LANES, FTXT), jnp.float32),
        pltpu.VMEM((LANES, FVIS), jnp.float32),
        pltpu.VMEM((LANES, FTXT), jnp.float32),
        pltpu.SemaphoreType.DMA,
        pltpu.SemaphoreType.DMA,
    ],
)(_sc_gather_body)


def kernel(inputs, text, W1, b1, W2, b2, W3, b3):
    scores, mags3, idx, sel = _scores_mags(inputs, text, W1, b1, W2, b2, W3,
                                           b3)
    mags = mags3[:, :, 0]
    score_normal = sel[:HALF, 0]
    score_abnormal = sel[HALF:, 0]

    vis2 = inputs.reshape(BS * NCROPS * T, FVIS)
    txt2 = text.reshape(BS * NCROPS * T, FTXT)
    feat_n, feat_a0, feat_a1, feat_a2, feat_a3, feat_a4 = _sc_gather(
        vis2, txt2, idx)

    def shape(f):
        return f.reshape(K, NCROPS * HALF, FFUSE).transpose(1, 0, 2)

    return (score_abnormal, score_normal, shape(feat_a0), shape(feat_n),
            shape(feat_a1), shape(feat_a2), scores, shape(feat_a3),
            shape(feat_a4), mags)
